# baseline (speedup 1.0000x reference)
import functools

import jax
import jax.numpy as jnp
from jax import lax
from jax.experimental import pallas as pl
from jax.experimental.pallas import tpu as pltpu
from jax.experimental.pallas import tpu_sc as plsc

BATCH = 1024
IN_DIM = 4096
OUT_DIM = 16384

NC = 2
NS = 16
NW = NC * NS

N_BLOCKS = 4
R_BLOCKS = NW // N_BLOCKS
NCHUNK = OUT_DIM // N_BLOCKS
ROWS_PER_TILE = BATCH // R_BLOCKS
RG = 16
NGROUPS = ROWS_PER_TILE // RG
OCW = 1024
NOC = NCHUNK // OCW
NVEC = OCW // 16


def _coeff_body(wt_ref, out_ref):
    w = wt_ref[...]
    m = jnp.max(w, axis=0, keepdims=True)
    e = jnp.exp(w - m)
    p = e / jnp.sum(e, axis=0, keepdims=True)

    def r(i):
        return p[i:i + 1, :]

    w0 = r(8) + r(9) + r(10) + r(11) + r(12) + r(13) + r(14) + r(15)
    wa = r(2) + r(3) + r(6) + r(7) - r(8) - r(9) - r(12) - r(13)
    wb = r(4) + r(5) + r(6) + r(7) - r(8) - r(9) - r(10) - r(11)
    wab = (r(1) - r(2) - r(4) - 2.0 * r(6) - r(7)
           + r(8) + 2.0 * r(9) + r(11) + r(13) - r(14))
    out_ref[...] = jnp.concatenate([w0, wa, wb, wab], axis=0)


_sc_mesh = plsc.VectorSubcoreMesh(core_axis_name="c", subcore_axis_name="s")


@functools.partial(
    pl.kernel,
    mesh=_sc_mesh,
    compiler_params=pltpu.CompilerParams(needs_layout_passes=False),
    out_type=jax.ShapeDtypeStruct((BATCH, OUT_DIM), jnp.float32),
    scratch_types=[
        pltpu.VMEM((2 * NCHUNK,), jnp.int32),
        pltpu.VMEM((4 * NCHUNK,), jnp.float32),
        pltpu.VMEM((RG * IN_DIM,), jnp.float32),
        pltpu.VMEM((2 * RG * OCW,), jnp.float32),
        pltpu.SemaphoreType.DMA,
        pltpu.SemaphoreType.DMA,
        pltpu.SemaphoreType.DMA,
    ],
)
def _sc_fn(x_hbm, idx_hbm, cf_hbm, out_hbm, idx_v, cf_v, x_v, o_v,
           sem_x, sem_o0, sem_o1):
    wid = lax.axis_index("s") * NC + lax.axis_index("c")
    nblk = wid % N_BLOCKS
    rblk = wid // N_BLOCKS
    n0 = nblk * NCHUNK
    row0 = rblk * ROWS_PER_TILE
    o_sems = (sem_o0, sem_o1)

    stage = []
    for j in range(2):
        stage.append(pltpu.make_async_copy(
            idx_hbm.at[j, pl.ds(n0, NCHUNK)],
            idx_v.at[pl.ds(j * NCHUNK, NCHUNK)], sem_x))
    for j in range(4):
        stage.append(pltpu.make_async_copy(
            cf_hbm.at[j, pl.ds(n0, NCHUNK)],
            cf_v.at[pl.ds(j * NCHUNK, NCHUNK)], sem_x))
    for c in stage:
        c.start()
    for c in stage:
        c.wait()

    def group_body(g, carry):
        r0 = row0 + g * RG
        with jax.named_scope("xload"):
            xcps = [pltpu.make_async_copy(
                x_hbm.at[r0 + rr, :],
                x_v.at[pl.ds(rr * IN_DIM, IN_DIM)], sem_x)
                for rr in range(RG)]
            for c in xcps:
                c.start()
            for c in xcps:
                c.wait()
        for oc in range(NOC):
            p = oc & 1
            col0 = n0 + oc * OCW

            def drain():
                for rr in range(RG):
                    pltpu.make_async_copy(
                        o_v.at[pl.ds((p * RG + rr) * OCW, OCW)],
                        out_hbm.at[r0 + rr, pl.ds(col0, OCW)],
                        o_sems[p]).wait()
            with jax.named_scope("drain"):
                if oc >= 2:
                    drain()
                else:
                    @pl.when(g > 0)
                    def _():
                        drain()

            @plsc.parallel_loop(0, NVEC, unroll=1)
            def vec_body(j):
                off = oc * OCW + j * 16
                ia = idx_v[pl.ds(off, 16)]
                ib = idx_v[pl.ds(NCHUNK + off, 16)]
                w0 = cf_v[pl.ds(off, 16)]
                wa = cf_v[pl.ds(NCHUNK + off, 16)]
                wb = cf_v[pl.ds(2 * NCHUNK + off, 16)]
                wab = cf_v[pl.ds(3 * NCHUNK + off, 16)]
                avs = [plsc.load_gather(x_v.at[pl.ds(rr * IN_DIM, IN_DIM)],
                                        [ia]) for rr in range(RG)]
                bvs = [plsc.load_gather(x_v.at[pl.ds(rr * IN_DIM, IN_DIM)],
                                        [ib]) for rr in range(RG)]
                for rr in range(RG):
                    a, b = avs[rr], bvs[rr]
                    o_v[pl.ds((p * RG + rr) * OCW + j * 16, 16)] = (
                        (wa + wab * b) * a + (wb * b + w0))
            for rr in range(RG):
                pltpu.make_async_copy(
                    o_v.at[pl.ds((p * RG + rr) * OCW, OCW)],
                    out_hbm.at[r0 + rr, pl.ds(col0, OCW)],
                    o_sems[p]).start()
        return carry

    lax.fori_loop(0, NGROUPS, group_body, 0)

    r_last = row0 + (NGROUPS - 1) * RG
    for oc in (NOC - 2, NOC - 1):
        p = oc & 1
        for rr in range(RG):
            pltpu.make_async_copy(
                o_v.at[pl.ds((p * RG + rr) * OCW, OCW)],
                out_hbm.at[r_last + rr, pl.ds(n0 + oc * OCW, OCW)],
                o_sems[p]).wait()


def kernel(x, weight, indices):
    coeffs = pl.pallas_call(
        _coeff_body,
        out_shape=jax.ShapeDtypeStruct((4, OUT_DIM), jnp.float32),
    )(weight.T)
    return _sc_fn(x, indices, coeffs)


# speedup vs baseline: 1.0709x; 1.0709x over previous
import functools

import jax
import jax.numpy as jnp
from jax import lax
from jax.experimental import pallas as pl
from jax.experimental.pallas import tpu as pltpu
from jax.experimental.pallas import tpu_sc as plsc

BATCH = 1024
IN_DIM = 4096
OUT_DIM = 16384

NC = 2
NS = 16
NW = NC * NS

N_BLOCKS = 2
R_BLOCKS = NW // N_BLOCKS
NCHUNK = OUT_DIM // N_BLOCKS
ROWS_PER_TILE = BATCH // R_BLOCKS
RG = 16
NGROUPS = ROWS_PER_TILE // RG
OCW = 512
NOC = NCHUNK // OCW
NVEC = OCW // 16


def _coeff_body(wt_ref, out_ref):
    w = wt_ref[...]
    m = jnp.max(w, axis=0, keepdims=True)
    e = jnp.exp(w - m)
    p = e / jnp.sum(e, axis=0, keepdims=True)

    def r(i):
        return p[i:i + 1, :]

    w0 = r(8) + r(9) + r(10) + r(11) + r(12) + r(13) + r(14) + r(15)
    wa = r(2) + r(3) + r(6) + r(7) - r(8) - r(9) - r(12) - r(13)
    wb = r(4) + r(5) + r(6) + r(7) - r(8) - r(9) - r(10) - r(11)
    wab = (r(1) - r(2) - r(4) - 2.0 * r(6) - r(7)
           + r(8) + 2.0 * r(9) + r(11) + r(13) - r(14))
    out_ref[...] = jnp.concatenate([w0, wa, wb, wab], axis=0)


_sc_mesh = plsc.VectorSubcoreMesh(core_axis_name="c", subcore_axis_name="s")


@functools.partial(
    pl.kernel,
    mesh=_sc_mesh,
    compiler_params=pltpu.CompilerParams(needs_layout_passes=False),
    out_type=jax.ShapeDtypeStruct((BATCH, OUT_DIM), jnp.float32),
    scratch_types=[
        pltpu.VMEM((NCHUNK,), jnp.int32),
        pltpu.VMEM((4 * NCHUNK,), jnp.float32),
        pltpu.VMEM((RG * IN_DIM,), jnp.float32),
        pltpu.VMEM((2 * RG * OCW,), jnp.float32),
        pltpu.SemaphoreType.DMA,
        pltpu.SemaphoreType.DMA,
        pltpu.SemaphoreType.DMA,
    ],
)
def _sc_fn(x_hbm, idx_hbm, cf_hbm, out_hbm, idx_v, cf_v, x_v, o_v,
           sem_x, sem_o0, sem_o1):
    wid = lax.axis_index("s") * NC + lax.axis_index("c")
    nblk = wid % N_BLOCKS
    rblk = wid // N_BLOCKS
    n0 = nblk * NCHUNK
    row0 = rblk * ROWS_PER_TILE
    o_sems = (sem_o0, sem_o1)

    stage = [pltpu.make_async_copy(
        idx_hbm.at[pl.ds(n0, NCHUNK)], idx_v, sem_x)]
    for j in range(4):
        stage.append(pltpu.make_async_copy(
            cf_hbm.at[j, pl.ds(n0, NCHUNK)],
            cf_v.at[pl.ds(j * NCHUNK, NCHUNK)], sem_x))
    for c in stage:
        c.start()
    for c in stage:
        c.wait()

    def group_body(g, carry):
        r0 = row0 + g * RG
        with jax.named_scope("xload"):
            xcps = [pltpu.make_async_copy(
                x_hbm.at[r0 + rr, :],
                x_v.at[pl.ds(rr * IN_DIM, IN_DIM)], sem_x)
                for rr in range(RG)]
            for c in xcps:
                c.start()
            for c in xcps:
                c.wait()
        for oc in range(NOC):
            p = oc & 1
            col0 = n0 + oc * OCW

            def drain():
                for rr in range(RG):
                    pltpu.make_async_copy(
                        o_v.at[pl.ds((p * RG + rr) * OCW, OCW)],
                        out_hbm.at[r0 + rr, pl.ds(col0, OCW)],
                        o_sems[p]).wait()
            with jax.named_scope("drain"):
                if oc >= 2:
                    drain()
                else:
                    @pl.when(g > 0)
                    def _():
                        drain()

            @plsc.parallel_loop(0, NVEC, unroll=1)
            def vec_body(j):
                off = oc * OCW + j * 16
                iv = idx_v[pl.ds(off, 16)]
                ia = iv & 0xFFFF
                ib = lax.shift_right_logical(iv, 16)
                w0 = cf_v[pl.ds(off, 16)]
                wa = cf_v[pl.ds(NCHUNK + off, 16)]
                wb = cf_v[pl.ds(2 * NCHUNK + off, 16)]
                wab = cf_v[pl.ds(3 * NCHUNK + off, 16)]
                avs = [plsc.load_gather(x_v.at[pl.ds(rr * IN_DIM, IN_DIM)],
                                        [ia]) for rr in range(RG)]
                bvs = [plsc.load_gather(x_v.at[pl.ds(rr * IN_DIM, IN_DIM)],
                                        [ib]) for rr in range(RG)]
                for rr in range(RG):
                    a, b = avs[rr], bvs[rr]
                    o_v[pl.ds((p * RG + rr) * OCW + j * 16, 16)] = (
                        (wa + wab * b) * a + (wb * b + w0))
            for rr in range(RG):
                pltpu.make_async_copy(
                    o_v.at[pl.ds((p * RG + rr) * OCW, OCW)],
                    out_hbm.at[r0 + rr, pl.ds(col0, OCW)],
                    o_sems[p]).start()
        return carry

    lax.fori_loop(0, NGROUPS, group_body, 0)

    r_last = row0 + (NGROUPS - 1) * RG
    for oc in (NOC - 2, NOC - 1):
        p = oc & 1
        for rr in range(RG):
            pltpu.make_async_copy(
                o_v.at[pl.ds((p * RG + rr) * OCW, OCW)],
                out_hbm.at[r_last + rr, pl.ds(n0 + oc * OCW, OCW)],
                o_sems[p]).wait()


def kernel(x, weight, indices):
    coeffs = pl.pallas_call(
        _coeff_body,
        out_shape=jax.ShapeDtypeStruct((4, OUT_DIM), jnp.float32),
    )(weight.T)
    packed_idx = indices[0] | (indices[1] << 16)
    return _sc_fn(x, packed_idx, coeffs)
